# baseline (speedup 1.0000x reference)
import functools

import jax
import jax.numpy as jnp
from jax import lax
from jax.experimental import pallas as pl
from jax.experimental.pallas import tpu as pltpu
from jax.experimental.pallas import tpu_sc as plsc

_N = 8192
_NT = 16
_CHUNK = _N // _NT
_L = 16
_NV = _CHUNK // _L

_CLASS = 42
_MIN = 0.1
_MAX = 1.0
_MATCHED_POINT = 1.5


@functools.partial(
    pl.kernel,
    out_type=jax.ShapeDtypeStruct((_L,), jnp.float32),
    mesh=plsc.VectorSubcoreMesh(
        core_axis_name="c", subcore_axis_name="s", num_cores=1
    ),
    scratch_types=[
        pltpu.VMEM((_CHUNK,), jnp.float32),
        pltpu.VMEM((_L,), jnp.float32),
        pltpu.VMEM((_L,), jnp.float32),
        pltpu.VMEM_SHARED((_NT, _L), jnp.float32),
        pltpu.VMEM((_NT, _L), jnp.float32),
        pltpu.VMEM((_L,), jnp.float32),
        pltpu.SemaphoreType.DMA,
    ],
)
def _judge_sc(score_hbm, out_hbm, chunk_v, cvec_v, acc_v, shared, all_v,
              res_v, sem):
    sid = lax.axis_index("s")
    base = sid * _CHUNK

    cp = pltpu.async_copy(score_hbm.at[0, pl.ds(base, _CHUNK)], chunk_v, sem)
    pltpu.sync_copy(score_hbm.at[0, pl.ds(40, _L)], cvec_v)

    lane = lax.iota(jnp.int32, _L)
    cv = cvec_v[...]
    cb = cv.at[jnp.full((_L,), _CLASS - 40, jnp.int32)].get(
        mode="promise_in_bounds")

    cp.wait()

    one = jnp.full((_L,), 1.0, jnp.float32)
    zero = jnp.zeros((_L,), jnp.float32)
    acc = jnp.zeros((_L,), jnp.float32)
    for i in range(_NV):
        v = chunk_v[pl.ds(i * _L, _L)]
        g = base + (i * _L) + lane
        gt = jnp.where(v > cb, one, zero)
        eq = jnp.where(v == cb, one, zero)
        lt42 = jnp.where(g < _CLASS, one, zero)
        acc = acc + gt + eq * lt42
    acc_v[...] = acc

    pltpu.sync_copy(acc_v, shared.at[sid])
    plsc.subcore_barrier()

    @pl.when(sid == 0)
    def _finish():
        pltpu.sync_copy(shared, all_v)
        tot = jnp.zeros((_L,), jnp.float32)
        for t in range(_NT):
            tot = tot + all_v[t]
        for sh in (8, 4, 2, 1):
            tot = tot + tot.at[(lane + sh) & (_L - 1)].get(
                mode="promise_in_bounds")
        rank = tot

        hc = jnp.where(rank < 10.0, one, zero)
        inr = (jnp.where(cb >= _MIN, one, zero)
               * jnp.where(cb <= _MAX, one, zero))
        jf = jnp.minimum(inr + hc, one)
        point = inr * _MATCHED_POINT + (one - inr) * hc
        res_v[...] = jnp.where(lane == 0, jf,
                               jnp.where(lane == 1, point, zero))
        pltpu.sync_copy(res_v, out_hbm)


def kernel(score):
    out = _judge_sc(score)
    return out[0] > 0.5, out[1]


# speedup vs baseline: 1.0374x; 1.0374x over previous
import functools

import jax
import jax.numpy as jnp
from jax import lax
from jax.experimental import pallas as pl
from jax.experimental.pallas import tpu as pltpu
from jax.experimental.pallas import tpu_sc as plsc

_N = 8192
_NT = 16
_CHUNK = _N // _NT
_L = 16
_NV = _CHUNK // _L

_CLASS = 42
_MIN = 0.1
_MAX = 1.0
_MATCHED_POINT = 1.5


@functools.partial(
    pl.kernel,
    out_type=jax.ShapeDtypeStruct((_L,), jnp.float32),
    mesh=plsc.VectorSubcoreMesh(
        core_axis_name="c", subcore_axis_name="s", num_cores=1
    ),
    scratch_types=[
        pltpu.VMEM((_CHUNK,), jnp.float32),
        pltpu.VMEM((_L,), jnp.float32),
        pltpu.VMEM((_L,), jnp.float32),
        pltpu.VMEM_SHARED((_NT, _L), jnp.float32),
        pltpu.VMEM((_NT, _L), jnp.float32),
        pltpu.VMEM((_L,), jnp.float32),
        pltpu.SemaphoreType.DMA,
    ],
)
def _judge_sc(score_hbm, out_hbm, chunk_v, cvec_v, acc_v, shared, all_v,
              res_v, sem):
    sid = lax.axis_index("s")
    base = sid * _CHUNK

    cp = pltpu.async_copy(score_hbm.at[0, pl.ds(base, _CHUNK)], chunk_v, sem)
    pltpu.sync_copy(score_hbm.at[0, pl.ds(40, _L)], cvec_v)

    lane = lax.iota(jnp.int32, _L)
    cv = cvec_v[...]
    cb = cv.at[jnp.full((_L,), _CLASS - 40, jnp.int32)].get(
        mode="promise_in_bounds")

    cp.wait()

    one = jnp.full((_L,), 1.0, jnp.float32)
    zero = jnp.zeros((_L,), jnp.float32)

    def _count(i, acc):
        v = chunk_v[pl.ds(i * _L, _L)]
        g = base + (i * _L) + lane
        gt = jnp.where(v > cb, one, zero)
        eq = jnp.where(v == cb, one, zero)
        lt42 = jnp.where(g < _CLASS, one, zero)
        return acc + gt + eq * lt42

    acc = lax.fori_loop(0, _NV, _count, jnp.zeros((_L,), jnp.float32))
    acc_v[...] = acc

    pltpu.sync_copy(acc_v, shared.at[sid])
    plsc.subcore_barrier()

    @pl.when(sid == 0)
    def _finish():
        pltpu.sync_copy(shared, all_v)
        tot = jnp.zeros((_L,), jnp.float32)
        for t in range(_NT):
            tot = tot + all_v[t]
        for sh in (8, 4, 2, 1):
            tot = tot + tot.at[(lane + sh) & (_L - 1)].get(
                mode="promise_in_bounds")
        rank = tot

        hc = jnp.where(rank < 10.0, one, zero)
        inr = (jnp.where(cb >= _MIN, one, zero)
               * jnp.where(cb <= _MAX, one, zero))
        point = inr * _MATCHED_POINT + (one - inr) * hc
        res_v[...] = jnp.where(lane == 0, point, zero)
        pltpu.sync_copy(res_v, out_hbm)


def kernel(score):
    out = _judge_sc(score)
    point = out[0]
    return point > 0.5, point


# speedup vs baseline: 1.0402x; 1.0027x over previous
import functools

import jax
import jax.numpy as jnp
from jax import lax
from jax.experimental import pallas as pl
from jax.experimental.pallas import tpu as pltpu
from jax.experimental.pallas import tpu_sc as plsc

_N = 8192
_NT = 16
_CHUNK = _N // _NT
_L = 16
_NV = _CHUNK // _L

_CLASS = 42
_MIN = 0.1
_MAX = 1.0
_MATCHED_POINT = 1.5


@functools.partial(
    pl.kernel,
    out_type=jax.ShapeDtypeStruct((_L,), jnp.float32),
    mesh=plsc.VectorSubcoreMesh(
        core_axis_name="c", subcore_axis_name="s", num_cores=1
    ),
    scratch_types=[
        pltpu.VMEM((_CHUNK + _L,), jnp.float32),
        pltpu.VMEM_SHARED((_NT * _L,), jnp.float32),
        pltpu.SemaphoreType.DMA,
    ],
)
def _judge_sc(score_hbm, out_hbm, buf_v, shared, sem):
    sid = lax.axis_index("s")
    base = sid * _CHUNK

    cp = pltpu.async_copy(
        score_hbm.at[0, pl.ds(base, _CHUNK)], buf_v.at[pl.ds(0, _CHUNK)], sem)
    pltpu.sync_copy(score_hbm.at[0, pl.ds(40, _L)],
                    buf_v.at[pl.ds(_CHUNK, _L)])

    lane = lax.iota(jnp.int32, _L)
    cv = buf_v[pl.ds(_CHUNK, _L)]
    cb = cv.at[jnp.full((_L,), _CLASS - 40, jnp.int32)].get(
        mode="promise_in_bounds")

    cp.wait()

    one = jnp.full((_L,), 1.0, jnp.float32)
    zero = jnp.zeros((_L,), jnp.float32)

    def _count(i, acc):
        v = buf_v[pl.ds(i * _L, _L)]
        g = base + (i * _L) + lane
        gt = jnp.where(v > cb, one, zero)
        eq = jnp.where(v == cb, one, zero)
        lt42 = jnp.where(g < _CLASS, one, zero)
        return acc + gt + eq * lt42

    acc = lax.fori_loop(0, _NV, _count, jnp.zeros((_L,), jnp.float32))
    buf_v[pl.ds(_CHUNK, _L)] = acc

    pltpu.sync_copy(buf_v.at[pl.ds(_CHUNK, _L)], shared.at[pl.ds(sid * _L, _L)])
    plsc.subcore_barrier()

    @pl.when(sid == 0)
    def _finish():
        pltpu.sync_copy(shared, buf_v.at[pl.ds(0, _NT * _L)])
        tot = jnp.zeros((_L,), jnp.float32)
        for t in range(_NT):
            tot = tot + buf_v[pl.ds(t * _L, _L)]
        for sh in (8, 4, 2, 1):
            tot = tot + tot.at[(lane + sh) & (_L - 1)].get(
                mode="promise_in_bounds")
        rank = tot

        hc = jnp.where(rank < 10.0, one, zero)
        inr = (jnp.where(cb >= _MIN, one, zero)
               * jnp.where(cb <= _MAX, one, zero))
        point = inr * _MATCHED_POINT + (one - inr) * hc
        buf_v[pl.ds(_CHUNK, _L)] = jnp.where(lane == 0, point, zero)
        pltpu.sync_copy(buf_v.at[pl.ds(_CHUNK, _L)], out_hbm)


def kernel(score):
    out = _judge_sc(score)
    point = out[0]
    return point > 0.5, point
